# baseline (speedup 1.0000x reference)
import functools

import jax
import jax.numpy as jnp
from jax import lax
from jax.experimental import pallas as pl
from jax.experimental.pallas import tpu as pltpu
from jax.experimental.pallas import tpu_sc as plsc

_NUM_WORKERS = 32
_CHUNK = 128
_NBUF = 10
_DEPTH = 5


def _emb_body(n_chunks, idx_hbm, table_hbm, out_hbm, idx_v, *rest):
    bufs = rest[:_NBUF]
    gsems = rest[_NBUF:2 * _NBUF]
    ssems = rest[2 * _NBUF:3 * _NBUF]

    wid = lax.axis_index("s") * 2 + lax.axis_index("c")
    cbase = wid * n_chunks
    rbase = cbase * _CHUNK
    pltpu.sync_copy(idx_hbm.at[pl.ds(cbase, n_chunks)], idx_v)

    def fire(c, b):
        pltpu.async_copy(table_hbm.at[idx_v.at[c]], bufs[b], gsems[b])

    def drain_gather(b):
        pltpu.make_async_copy(
            out_hbm.at[pl.ds(0, _CHUNK)], bufs[b], gsems[b]).wait()

    def store(c, b):
        pltpu.async_copy(
            bufs[b], out_hbm.at[pl.ds(rbase + c * _CHUNK, _CHUNK)], ssems[b])

    def drain_store(b):
        pltpu.make_async_copy(
            bufs[b], out_hbm.at[pl.ds(0, _CHUNK)], ssems[b]).wait()

    for b in range(_DEPTH):
        fire(b, b)

    @pl.loop(0, n_chunks, step=_NBUF)
    def _(ci):
        for b in range(_NBUF):
            c = ci + b
            s = (b + _DEPTH) % _NBUF

            @pl.when(c + _DEPTH < n_chunks)
            def _():
                @pl.when(c + _DEPTH >= _NBUF)
                def _():
                    drain_store(s)

                fire(c + _DEPTH, s)

            drain_gather(b)
            store(c, b)

    for b in range(_NBUF):
        drain_store(b)


def kernel(inputs, embeddings):
    b, s = inputs.shape
    v, d = embeddings.shape
    n_flat = b * s
    n_chunks = n_flat // _CHUNK // _NUM_WORKERS
    idx = inputs.astype(jnp.int32).reshape(n_flat // _CHUNK, _CHUNK)

    mesh = plsc.VectorSubcoreMesh(core_axis_name="c", subcore_axis_name="s")
    emb = pl.kernel(
        functools.partial(_emb_body, n_chunks),
        out_type=jax.ShapeDtypeStruct((n_flat, d), jnp.float32),
        mesh=mesh,
        scratch_types=[pltpu.VMEM((n_chunks, _CHUNK), jnp.int32)]
        + [pltpu.VMEM((_CHUNK, d), jnp.float32)] * _NBUF
        + [pltpu.SemaphoreType.DMA] * (2 * _NBUF),
        compiler_params=pltpu.CompilerParams(use_tc_tiling_on_sc=False),
    )
    return emb(idx, embeddings).reshape(b, s, d)


# speedup vs baseline: 1.6229x; 1.6229x over previous
import functools

import jax
import jax.numpy as jnp
from jax import lax
from jax.experimental import pallas as pl
from jax.experimental.pallas import tpu as pltpu
from jax.experimental.pallas import tpu_sc as plsc

_NUM_WORKERS = 32
_GRP = 16


def _emb_body(n_groups, rows_per_worker, idx_hbm, table_hbm, out_hbm,
              idx_v, buf_a, buf_b, gsem_a, gsem_b, ssem_a, ssem_b):
    wid = lax.axis_index("s") * 2 + lax.axis_index("c")
    base = wid * rows_per_worker
    pltpu.sync_copy(idx_hbm.at[pl.ds(base, rows_per_worker)], idx_v)

    def fire_gathers(g, buf, sem):
        for r in range(_GRP):
            pltpu.async_copy(
                table_hbm.at[idx_v.at[g * _GRP + r]],
                buf.at[r], sem)

    def drain_gathers(buf, sem):
        pltpu.make_async_copy(
            out_hbm.at[pl.ds(0, _GRP)], buf, sem).wait()

    def store_rows(g, buf, sem):
        return pltpu.async_copy(
            buf, out_hbm.at[pl.ds(base + g * _GRP, _GRP)], sem)

    fire_gathers(0, buf_a, gsem_a)

    @pl.loop(0, n_groups, step=2)
    def _(gi):
        fire_gathers(gi + 1, buf_b, gsem_b)
        drain_gathers(buf_a, gsem_a)
        store_rows(gi, buf_a, ssem_a).wait()

        @pl.when(gi + 2 < n_groups)
        def _():
            fire_gathers(gi + 2, buf_a, gsem_a)

        drain_gathers(buf_b, gsem_b)
        store_rows(gi + 1, buf_b, ssem_b).wait()


def kernel(inputs, embeddings):
    b, s = inputs.shape
    v, d = embeddings.shape
    rows_per_worker = b // _NUM_WORKERS
    n_groups = rows_per_worker // _GRP
    idx = inputs.astype(jnp.int32)

    mesh = plsc.VectorSubcoreMesh(core_axis_name="c", subcore_axis_name="s")
    emb = pl.kernel(
        functools.partial(_emb_body, n_groups, rows_per_worker),
        out_type=jax.ShapeDtypeStruct((b, s, d), jnp.float32),
        mesh=mesh,
        scratch_types=[
            pltpu.VMEM((rows_per_worker, s), jnp.int32),
            pltpu.VMEM((_GRP, s, d), jnp.float32),
            pltpu.VMEM((_GRP, s, d), jnp.float32),
            pltpu.SemaphoreType.DMA,
            pltpu.SemaphoreType.DMA,
            pltpu.SemaphoreType.DMA,
            pltpu.SemaphoreType.DMA,
        ],
        compiler_params=pltpu.CompilerParams(use_tc_tiling_on_sc=False),
    )
    return emb(idx, embeddings)
